# Initial kernel scaffold; baseline (speedup 1.0000x reference)
import jax
import jax.numpy as jnp
from jax.experimental import pallas as pl


def kernel(x1, x2, x3, x4):
    raise NotImplementedError("write your pallas kernel here")


# speedup vs baseline: 2.8414x; 2.8414x over previous
import functools

import jax
import jax.numpy as jnp
from jax import lax
from jax.experimental import pallas as pl
from jax.experimental.pallas import tpu as pltpu
from jax.experimental.pallas import tpu_sc as plsc

N = 1048576
NUM_CORES = 2
NUM_SUBCORES = 16
NUM_WORKERS = NUM_CORES * NUM_SUBCORES
PER_WORKER = N // NUM_WORKERS
CHUNK = 8192
NUM_CHUNKS = PER_WORKER // CHUNK
LANES = 16


def _body(x1h, x2h, x3h, x4h, outh, b1, b2, b3, b4):
    wid = lax.axis_index("s") * NUM_CORES + lax.axis_index("c")
    base = wid * PER_WORKER

    def chunk_body(ci, carry):
        off = base + ci * CHUNK
        pltpu.sync_copy(x1h.at[pl.ds(off, CHUNK)], b1)
        pltpu.sync_copy(x2h.at[pl.ds(off, CHUNK)], b2)
        pltpu.sync_copy(x3h.at[pl.ds(off, CHUNK)], b3)
        pltpu.sync_copy(x4h.at[pl.ds(off, CHUNK)], b4)

        def inner(i, c):
            s = i * LANES
            a = b1[pl.ds(s, LANES)]
            b = b2[pl.ds(s, LANES)]
            cc = b3[pl.ds(s, LANES)]
            d = b4[pl.ds(s, LANES)]
            lo1 = jnp.minimum(a, b)
            hi1 = jnp.maximum(a, b)
            lo2 = jnp.minimum(cc, d)
            hi2 = jnp.maximum(cc, d)
            o0 = jnp.minimum(lo1, lo2)
            o3 = jnp.maximum(hi1, hi2)
            m1 = jnp.maximum(lo1, lo2)
            m2 = jnp.minimum(hi1, hi2)
            o1 = jnp.minimum(m1, m2)
            o2 = jnp.maximum(m1, m2)
            b1[pl.ds(s, LANES)] = o0
            b2[pl.ds(s, LANES)] = o1
            b3[pl.ds(s, LANES)] = o2
            b4[pl.ds(s, LANES)] = o3
            return c

        lax.fori_loop(0, CHUNK // LANES, inner, 0)

        pltpu.sync_copy(b1, outh.at[0, pl.ds(off, CHUNK)])
        pltpu.sync_copy(b2, outh.at[1, pl.ds(off, CHUNK)])
        pltpu.sync_copy(b3, outh.at[2, pl.ds(off, CHUNK)])
        pltpu.sync_copy(b4, outh.at[3, pl.ds(off, CHUNK)])
        return carry

    lax.fori_loop(0, NUM_CHUNKS, chunk_body, 0)


def kernel(x1, x2, x3, x4):
    mesh = plsc.VectorSubcoreMesh(core_axis_name="c", subcore_axis_name="s")
    run = functools.partial(
        pl.kernel,
        mesh=mesh,
        out_type=jax.ShapeDtypeStruct((4, N), jnp.float32),
        scratch_types=[
            pltpu.VMEM((CHUNK,), jnp.float32),
            pltpu.VMEM((CHUNK,), jnp.float32),
            pltpu.VMEM((CHUNK,), jnp.float32),
            pltpu.VMEM((CHUNK,), jnp.float32),
        ],
    )(_body)
    return run(x1, x2, x3, x4)


# speedup vs baseline: 3.7808x; 1.3306x over previous
import functools

import jax
import jax.numpy as jnp
from jax import lax
from jax.experimental import pallas as pl
from jax.experimental.pallas import tpu as pltpu
from jax.experimental.pallas import tpu_sc as plsc

N = 1048576
NUM_CORES = 2
NUM_SUBCORES = 16
NUM_WORKERS = NUM_CORES * NUM_SUBCORES
PER_WORKER = N // NUM_WORKERS
CHUNK = 8192
NUM_CHUNKS = PER_WORKER // CHUNK
LANES = 16


def _sort_chunk_inplace(bufs):
    def inner(i, c):
        s = i * LANES
        a = bufs[0][pl.ds(s, LANES)]
        b = bufs[1][pl.ds(s, LANES)]
        cc = bufs[2][pl.ds(s, LANES)]
        d = bufs[3][pl.ds(s, LANES)]
        lo1 = jnp.minimum(a, b)
        hi1 = jnp.maximum(a, b)
        lo2 = jnp.minimum(cc, d)
        hi2 = jnp.maximum(cc, d)
        o0 = jnp.minimum(lo1, lo2)
        o3 = jnp.maximum(hi1, hi2)
        m1 = jnp.maximum(lo1, lo2)
        m2 = jnp.minimum(hi1, hi2)
        bufs[0][pl.ds(s, LANES)] = o0
        bufs[1][pl.ds(s, LANES)] = jnp.minimum(m1, m2)
        bufs[2][pl.ds(s, LANES)] = jnp.maximum(m1, m2)
        bufs[3][pl.ds(s, LANES)] = o3
        return c

    lax.fori_loop(0, CHUNK // LANES, inner, 0)


def _body(x1h, x2h, x3h, x4h, outh,
          b00, b01, b02, b03, b10, b11, b12, b13,
          insem0, insem1, outsem0, outsem1):
    wid = lax.axis_index("s") * NUM_CORES + lax.axis_index("c")
    base = wid * PER_WORKER
    xs = (x1h, x2h, x3h, x4h)
    bufs = ((b00, b01, b02, b03), (b10, b11, b12, b13))
    insems = (insem0, insem1)
    outsems = (outsem0, outsem1)

    in_handles = [None, None]
    out_handles = [None, None]

    def start_inputs(ci):
        st = ci & 1
        off = base + ci * CHUNK
        in_handles[st] = [
            pltpu.async_copy(xs[j].at[pl.ds(off, CHUNK)], bufs[st][j], insems[st])
            for j in range(4)
        ]

    def start_outputs(ci):
        st = ci & 1
        off = base + ci * CHUNK
        out_handles[st] = [
            pltpu.async_copy(bufs[st][j], outh.at[j, pl.ds(off, CHUNK)], outsems[st])
            for j in range(4)
        ]

    start_inputs(0)
    for ci in range(NUM_CHUNKS):
        st = ci & 1
        if ci + 1 < NUM_CHUNKS:
            if ci >= 1:
                for h in out_handles[1 - st]:
                    h.wait()
            start_inputs(ci + 1)
        for h in in_handles[st]:
            h.wait()
        _sort_chunk_inplace(bufs[st])
        start_outputs(ci)
    for st in (0, 1):
        if out_handles[st] is not None:
            for h in out_handles[st]:
                h.wait()


def kernel(x1, x2, x3, x4):
    mesh = plsc.VectorSubcoreMesh(core_axis_name="c", subcore_axis_name="s")
    run = functools.partial(
        pl.kernel,
        mesh=mesh,
        out_type=jax.ShapeDtypeStruct((4, N), jnp.float32),
        scratch_types=(
            [pltpu.VMEM((CHUNK,), jnp.float32) for _ in range(8)]
            + [pltpu.SemaphoreType.DMA for _ in range(4)]
        ),
    )(_body)
    return run(x1, x2, x3, x4)
